# baseline (speedup 1.0000x reference)
import dataclasses
import functools

import jax
import jax.numpy as jnp
import numpy as np
from jax import lax
from jax.experimental import pallas as pl
from jax.experimental.pallas import tpu as pltpu
from jax.experimental.pallas import tpu_sc as plsc

EMBED_D = 32
LANES = 16
NUM_CORES = 2
NUM_SUBCORES = 16
NUM_WORKERS = NUM_CORES * NUM_SUBCORES
CHUNK = 640
PITCH = CHUNK + 1
IDX_TILE = 128
GATHERS_PER_CHUNK = CHUNK // IDX_TILE

_MAGIC = 0x5F3759DF


def _rsqrt_scale(s):
    bits = plsc.bitcast(s, jnp.int32)
    y = plsc.bitcast(np.int32(_MAGIC) - (bits >> 1), jnp.float32)
    y = y * (1.5 - 0.5 * s * y * y)
    y = y * (1.5 - 0.5 * s * y * y)
    return jnp.where(s > 9.0, 3.0 * y, 1.0)


_SHUFFLE_DNUMS = lax.GatherDimensionNumbers(
    offset_dims=(), collapsed_slice_dims=(0,), start_index_map=(0,)
)


def _lane_shuffle(v, idx):
    return lax.gather(
        v,
        idx[:, None],
        dimension_numbers=_SHUFFLE_DNUMS,
        slice_sizes=(1,),
        mode=lax.GatherScatterMode.PROMISE_IN_BOUNDS,
    )


def _renorm_row_t(rows, obuf, r):
    lane = lax.iota(jnp.int32, LANES)
    a = rows[r, pl.ds(0, LANES)]
    b = rows[r, pl.ds(LANES, LANES)]
    s = a * a + b * b
    for p in (8, 4, 2, 1):
        s = s + _lane_shuffle(s, lane ^ p)
    scale = _rsqrt_scale(s)
    col = jnp.full((LANES,), r, jnp.int32)
    plsc.store_scatter(obuf, [lane, col], a * scale)
    plsc.store_scatter(obuf, [lane + LANES, col], b * scale)


def _renorm_chunk_t(rows, obuf):
    @pl.loop(0, CHUNK, step=4)
    def _(r):
        for u in range(4):
            _renorm_row_t(rows, obuf, r + u)


def _make_sc_lookup(n_rows):
    rows_per_w = n_rows // NUM_WORKERS
    n_chunks = rows_per_w // CHUNK
    idx_rows_per_w = rows_per_w // IDX_TILE
    assert rows_per_w % CHUNK == 0 and n_chunks % 2 == 0

    mesh = plsc.VectorSubcoreMesh(core_axis_name="c", subcore_axis_name="s")

    cp = pltpu.CompilerParams()
    if "needs_layout_passes" in pltpu.CompilerParams.__dataclass_fields__:
        cp = dataclasses.replace(cp, needs_layout_passes=False)
    if "use_tc_tiling_on_sc" in pltpu.CompilerParams.__dataclass_fields__:
        cp = dataclasses.replace(cp, use_tc_tiling_on_sc=False)

    @functools.partial(
        pl.kernel,
        out_type=jax.ShapeDtypeStruct((EMBED_D, n_rows), jnp.float32),
        mesh=mesh,
        compiler_params=cp,
        scratch_types=[
            pltpu.VMEM((idx_rows_per_w, IDX_TILE), jnp.int32),
            pltpu.VMEM((CHUNK, EMBED_D), jnp.float32),
            pltpu.VMEM((CHUNK, EMBED_D), jnp.float32),
            pltpu.VMEM((EMBED_D, PITCH), jnp.float32),
            pltpu.VMEM((EMBED_D, PITCH), jnp.float32),
            pltpu.SemaphoreType.DMA,
            pltpu.SemaphoreType.DMA,
            pltpu.SemaphoreType.DMA,
            pltpu.SemaphoreType.DMA,
        ],
    )
    def sc_lookup(
        idx_hbm, table_hbm, out_hbm,
        idx_all, rows0, rows1, obuf0, obuf1, gs0, gs1, os0, os1,
    ):
        wid = lax.axis_index("s") * NUM_CORES + lax.axis_index("c")
        row_base = wid * rows_per_w
        irow_base = wid * idx_rows_per_w

        pltpu.sync_copy(idx_hbm.at[pl.ds(irow_base, idx_rows_per_w)], idx_all)

        def start_gather(rows_buf, sem, c):
            for k in range(GATHERS_PER_CHUNK):
                pltpu.async_copy(
                    table_hbm.at[idx_all.at[c * GATHERS_PER_CHUNK + k]],
                    rows_buf.at[pl.ds(k * IDX_TILE, IDX_TILE)],
                    sem,
                )

        def wait_gather(rows_buf, sem, c):
            for k in range(GATHERS_PER_CHUNK):
                pltpu.make_async_copy(
                    table_hbm.at[idx_all.at[c * GATHERS_PER_CHUNK + k]],
                    rows_buf.at[pl.ds(k * IDX_TILE, IDX_TILE)],
                    sem,
                ).wait()

        def start_out(obuf, sem, c):
            pltpu.async_copy(
                obuf.at[:, pl.ds(0, CHUNK)],
                out_hbm.at[:, pl.ds(row_base + c * CHUNK, CHUNK)],
                sem,
            )

        def wait_out(obuf, sem, c):
            pltpu.make_async_copy(
                obuf.at[:, pl.ds(0, CHUNK)],
                out_hbm.at[:, pl.ds(row_base + c * CHUNK, CHUNK)],
                sem,
            ).wait()

        start_gather(rows0, gs0, 0)

        @pl.loop(0, n_chunks, step=2)
        def _(c):
            start_gather(rows1, gs1, c + 1)
            wait_gather(rows0, gs0, c)

            @pl.when(c >= 2)
            def _():
                wait_out(obuf0, os0, c - 2)

            _renorm_chunk_t(rows0, obuf0)
            start_out(obuf0, os0, c)

            @pl.when(c + 2 < n_chunks)
            def _():
                start_gather(rows0, gs0, c + 2)

            wait_gather(rows1, gs1, c + 1)

            @pl.when(c >= 2)
            def _():
                wait_out(obuf1, os1, c - 1)

            _renorm_chunk_t(rows1, obuf1)
            start_out(obuf1, os1, c + 1)

        wait_out(obuf0, os0, n_chunks - 2)
        wait_out(obuf1, os1, n_chunks - 1)

    return sc_lookup


def kernel(nouns_idx_tensor, conceptnet_embeddings):
    b, l = nouns_idx_tensor.shape
    n_rows = b * l
    idx2d = nouns_idx_tensor.reshape(n_rows // IDX_TILE, IDX_TILE).astype(jnp.int32)
    out_t = _make_sc_lookup(n_rows)(idx2d, conceptnet_embeddings)
    return jnp.transpose(out_t).reshape(b, l, EMBED_D)


# speedup vs baseline: 2.5866x; 2.5866x over previous
import dataclasses
import functools

import jax
import jax.numpy as jnp
import numpy as np
from jax import lax
from jax.experimental import pallas as pl
from jax.experimental.pallas import tpu as pltpu
from jax.experimental.pallas import tpu_sc as plsc

EMBED_D = 32
LANES = 16
NUM_CORES = 2
NUM_SUBCORES = 16
NUM_WORKERS = NUM_CORES * NUM_SUBCORES
CHUNK = 1280
IDX_TILE = 128
GATHERS_PER_CHUNK = CHUNK // IDX_TILE

_MAGIC = 0x5F3759DF


def _rsqrt_scale(s):
    bits = plsc.bitcast(s, jnp.int32)
    y = plsc.bitcast(np.int32(_MAGIC) - (bits >> 1), jnp.float32)
    y = y * (1.5 - 0.5 * s * y * y)
    y = y * (1.5 - 0.5 * s * y * y)
    return jnp.where(s > 9.0, 3.0 * y, 1.0)


_SHUFFLE_DNUMS = lax.GatherDimensionNumbers(
    offset_dims=(), collapsed_slice_dims=(0,), start_index_map=(0,)
)


def _lane_shuffle(v, idx):
    return lax.gather(
        v,
        idx[:, None],
        dimension_numbers=_SHUFFLE_DNUMS,
        slice_sizes=(1,),
        mode=lax.GatherScatterMode.PROMISE_IN_BOUNDS,
    )


def _renorm_pair(rows, r):
    lane = lax.iota(jnp.int32, LANES)
    a0 = rows[r, pl.ds(0, LANES)]
    b0 = rows[r, pl.ds(LANES, LANES)]
    a1 = rows[r + 1, pl.ds(0, LANES)]
    b1 = rows[r + 1, pl.ds(LANES, LANES)]
    s0 = a0 * a0 + b0 * b0
    s1 = a1 * a1 + b1 * b1
    t0 = s0 + _lane_shuffle(s0, lane ^ 8)
    t1 = s1 + _lane_shuffle(s1, lane ^ 8)
    m = jnp.where(lane < 8, t0, t1)
    for p in (4, 2, 1):
        m = m + _lane_shuffle(m, lane ^ p)
    scale = _rsqrt_scale(m)
    sc0 = _lane_shuffle(scale, jnp.zeros((LANES,), jnp.int32))
    sc1 = _lane_shuffle(scale, jnp.full((LANES,), 8, jnp.int32))
    rows[r, pl.ds(0, LANES)] = a0 * sc0
    rows[r, pl.ds(LANES, LANES)] = b0 * sc0
    rows[r + 1, pl.ds(0, LANES)] = a1 * sc1
    rows[r + 1, pl.ds(LANES, LANES)] = b1 * sc1


def _renorm_chunk(rows):
    @plsc.parallel_loop(0, CHUNK, 4, unroll=2)
    def _(r):
        _renorm_pair(rows, r)
        _renorm_pair(rows, r + 2)


def _make_sc_lookup(n_rows):
    rows_per_w = n_rows // NUM_WORKERS
    n_chunks = rows_per_w // CHUNK
    idx_rows_per_w = rows_per_w // IDX_TILE
    assert rows_per_w % CHUNK == 0 and n_chunks % 2 == 0

    mesh = plsc.VectorSubcoreMesh(core_axis_name="c", subcore_axis_name="s")

    cp = pltpu.CompilerParams()
    if "needs_layout_passes" in pltpu.CompilerParams.__dataclass_fields__:
        cp = dataclasses.replace(cp, needs_layout_passes=False)
    if "use_tc_tiling_on_sc" in pltpu.CompilerParams.__dataclass_fields__:
        cp = dataclasses.replace(cp, use_tc_tiling_on_sc=False)
    if "disable_bounds_checks" in pltpu.CompilerParams.__dataclass_fields__:
        cp = dataclasses.replace(cp, disable_bounds_checks=True)

    @functools.partial(
        pl.kernel,
        out_type=jax.ShapeDtypeStruct((n_rows, EMBED_D), jnp.float32),
        mesh=mesh,
        compiler_params=cp,
        scratch_types=[
            pltpu.VMEM((idx_rows_per_w, IDX_TILE), jnp.int32),
            pltpu.VMEM((CHUNK, EMBED_D), jnp.float32),
            pltpu.VMEM((CHUNK, EMBED_D), jnp.float32),
            pltpu.SemaphoreType.DMA,
            pltpu.SemaphoreType.DMA,
            pltpu.SemaphoreType.DMA,
            pltpu.SemaphoreType.DMA,
        ],
    )
    def sc_lookup(
        idx_hbm, table_hbm, out_hbm, idx_all, rows0, rows1, gs0, gs1, os0, os1
    ):
        wid = lax.axis_index("s") * NUM_CORES + lax.axis_index("c")
        row_base = wid * rows_per_w
        irow_base = wid * idx_rows_per_w

        pltpu.sync_copy(idx_hbm.at[pl.ds(irow_base, idx_rows_per_w)], idx_all)

        def start_gather(rows_buf, sem, c):
            for k in range(GATHERS_PER_CHUNK):
                pltpu.async_copy(
                    table_hbm.at[idx_all.at[c * GATHERS_PER_CHUNK + k]],
                    rows_buf.at[pl.ds(k * IDX_TILE, IDX_TILE)],
                    sem,
                )

        def wait_gather(rows_buf, sem, c):
            for k in range(GATHERS_PER_CHUNK):
                pltpu.make_async_copy(
                    table_hbm.at[idx_all.at[c * GATHERS_PER_CHUNK + k]],
                    rows_buf.at[pl.ds(k * IDX_TILE, IDX_TILE)],
                    sem,
                ).wait()

        def start_out(rows_buf, sem, c):
            pltpu.async_copy(
                rows_buf, out_hbm.at[pl.ds(row_base + c * CHUNK, CHUNK)], sem
            )

        def wait_out(rows_buf, sem, c):
            pltpu.make_async_copy(
                rows_buf, out_hbm.at[pl.ds(row_base + c * CHUNK, CHUNK)], sem
            ).wait()

        start_gather(rows0, gs0, 0)

        @pl.loop(0, n_chunks, step=2)
        def _(c):
            @pl.when(c >= 1)
            def _():
                wait_out(rows1, os1, c - 1)

            start_gather(rows1, gs1, c + 1)
            wait_gather(rows0, gs0, c)
            _renorm_chunk(rows0)
            start_out(rows0, os0, c)

            @pl.when(c + 2 < n_chunks)
            def _():
                wait_out(rows0, os0, c)
                start_gather(rows0, gs0, c + 2)

            wait_gather(rows1, gs1, c + 1)
            _renorm_chunk(rows1)
            start_out(rows1, os1, c + 1)

        wait_out(rows0, os0, n_chunks - 2)
        wait_out(rows1, os1, n_chunks - 1)

    return sc_lookup


def kernel(nouns_idx_tensor, conceptnet_embeddings):
    b, l = nouns_idx_tensor.shape
    n_rows = b * l
    idx2d = nouns_idx_tensor.reshape(n_rows // IDX_TILE, IDX_TILE).astype(jnp.int32)
    out = _make_sc_lookup(n_rows)(idx2d, conceptnet_embeddings)
    return out.reshape(b, l, EMBED_D)
